# baseline (speedup 1.0000x reference)
import functools

import jax
import jax.numpy as jnp
from jax import lax
from jax.experimental import pallas as pl
from jax.experimental.pallas import tpu as pltpu
from jax.experimental.pallas import tpu_sc as plsc

_B = 16384
_N_FIELDS = 26
_VOCAB = 100000
_K = 16
_TOTAL = _N_FIELDS * _VOCAB

_NC = 2
_NS = 16
_NW = _NC * _NS
_ROWS_PER_W = _B // _NW
_CHUNK_ROWS = 64
_IDX_PER_CHUNK = _CHUNK_ROWS * _N_FIELDS
_IDX_VECS = _IDX_PER_CHUNK // 128
_NCHUNKS = _ROWS_PER_W // _CHUNK_ROWS
_GROUPS = _CHUNK_ROWS // 16


def _fm_body(idx_hbm, quad_hbm, lin_hbm, outq_hbm, outl_hbm,
             idx_v, qrows_v, lrows_v, outq_v, outl_v, qsem, lsem):
    wid = lax.axis_index("s") * _NC + lax.axis_index("c")

    pltpu.sync_copy(idx_hbm.at[pl.ds(wid * (_NCHUNKS * _IDX_VECS),
                                     _NCHUNKS * _IDX_VECS)], idx_v)

    for c in range(_NCHUNKS):
        qcopies = [
            pltpu.async_copy(quad_hbm.at[idx_v.at[c * _IDX_VECS + j]],
                             qrows_v.at[pl.ds(j * 128, 128)], qsem)
            for j in range(_IDX_VECS)
        ]
        lcopies = [
            pltpu.async_copy(lin_hbm.at[idx_v.at[c * _IDX_VECS + j]],
                             lrows_v.at[pl.ds(j * 128, 128)], lsem)
            for j in range(_IDX_VECS)
        ]
        for cp in qcopies:
            cp.wait()
        for cp in lcopies:
            cp.wait()

        def group_body(g, _, c=c):
            rloc = g * 16 + lax.iota(jnp.int32, 16)
            ebase = rloc * _N_FIELDS
            zeros16 = jnp.zeros((16,), jnp.int32)

            lacc = jnp.zeros((16,), jnp.float32)
            for f in range(_N_FIELDS):
                lacc = lacc + plsc.load_gather(lrows_v, [ebase + f])

            def k_body(k, qacc):
                kvec = jnp.full((16,), k, jnp.int32)
                s = jnp.zeros((16,), jnp.float32)
                ss = jnp.zeros((16,), jnp.float32)
                for f in range(_N_FIELDS):
                    v = plsc.load_gather(qrows_v, [ebase + f, kvec])
                    s = s + v
                    ss = ss + v * v
                return qacc + (s * s - ss)

            qacc = lax.fori_loop(0, _K, k_body, jnp.zeros((16,), jnp.float32))

            off = c * _CHUNK_ROWS + g * 16
            outq_v[pl.ds(off, 16)] = 0.5 * qacc
            outl_v[pl.ds(off, 16)] = lacc
            return 0

        lax.fori_loop(0, _GROUPS, group_body, 0)

    out_base = wid * _ROWS_PER_W
    pltpu.sync_copy(outq_v, outq_hbm.at[pl.ds(out_base, _ROWS_PER_W)])
    pltpu.sync_copy(outl_v, outl_hbm.at[pl.ds(out_base, _ROWS_PER_W)])


@jax.jit
def _fm_sc(idx2d, quad_table, lin_table):
    mesh = plsc.VectorSubcoreMesh(core_axis_name="c", subcore_axis_name="s")
    run = functools.partial(
        pl.kernel,
        out_type=[
            jax.ShapeDtypeStruct((_B,), jnp.float32),
            jax.ShapeDtypeStruct((_B,), jnp.float32),
        ],
        mesh=mesh,
        compiler_params=pltpu.CompilerParams(use_tc_tiling_on_sc=False,
                                             needs_layout_passes=False),
        scratch_types=[
            pltpu.VMEM((_NCHUNKS * _IDX_VECS, 128), jnp.int32),
            pltpu.VMEM((_IDX_PER_CHUNK, _K), jnp.float32),
            pltpu.VMEM((_IDX_PER_CHUNK,), jnp.float32),
            pltpu.VMEM((_ROWS_PER_W,), jnp.float32),
            pltpu.VMEM((_ROWS_PER_W,), jnp.float32),
            pltpu.SemaphoreType.DMA,
            pltpu.SemaphoreType.DMA,
        ],
    )(_fm_body)
    return run(idx2d, quad_table, lin_table)


def kernel(input, quad_table, lin_table, global_bias):
    offsets = (jnp.arange(_N_FIELDS, dtype=jnp.int32) * _VOCAB)
    idx = input.astype(jnp.int32) + offsets[None, :]
    idx2d = idx.reshape(-1, 128)
    quadratic, linear = _fm_sc(idx2d, quad_table, lin_table.reshape(-1))
    bias = jnp.broadcast_to(global_bias, (input.shape[0],))
    return (quadratic, linear, bias)
